# Initial kernel scaffold; baseline (speedup 1.0000x reference)
import jax
import jax.numpy as jnp
from jax.experimental import pallas as pl


def kernel(x, W1, b1, W2, b2, W3, b3, W4, b4):
    raise NotImplementedError("write your pallas kernel here")


# speedup vs baseline: 3.5441x; 3.5441x over previous
import functools

import jax
import jax.numpy as jnp
from jax import lax
from jax.experimental import pallas as pl
from jax.experimental.pallas import tpu as pltpu
from jax.experimental.pallas import tpu_sc as plsc

_BINS = 128
_HEIGHT_RATE = 0.5


def _sc_histogram(xflat, B, F):
    info = plsc.get_sparse_core_info()
    NC, NS, L = info.num_cores, info.num_subcores, info.num_lanes
    NW = NC * NS
    SPW = B // NW
    CH = 49152
    NCH = F // CH
    assert B % NW == 0 and F % CH == 0 and CH % L == 0

    mesh = plsc.VectorSubcoreMesh(core_axis_name="c", subcore_axis_name="s")

    @functools.partial(
        pl.kernel,
        out_type=jax.ShapeDtypeStruct((B, _BINS), jnp.float32),
        mesh=mesh,
        compiler_params=pltpu.CompilerParams(needs_layout_passes=False),
        scratch_types=[
            pltpu.VMEM((_BINS * L,), jnp.float32),
            pltpu.VMEM((_BINS,), jnp.float32),
            pltpu.VMEM((CH,), jnp.float32),
            pltpu.VMEM((CH,), jnp.float32),
            pltpu.SemaphoreType.DMA,
            pltpu.SemaphoreType.DMA,
        ],
    )
    def hist_kernel(x_hbm, out_hbm, hist_v, row_v, buf0, buf1, sem0, sem1):
        wid = lax.axis_index("s") * NC + lax.axis_index("c")
        lane = lax.iota(jnp.int32, L)
        ones = jnp.ones((L,), jnp.float32)
        zeros = jnp.zeros((L,), jnp.float32)
        bufs = (buf0, buf1)
        sems = (sem0, sem1)

        for si in range(SPW):
            b = wid * SPW + si
            base = b * F

            def _zero(j, _):
                hist_v[pl.ds(j * L, L)] = zeros
                return 0
            lax.fori_loop(0, _BINS, _zero, 0)

            handles = [None, None]
            handles[0] = pltpu.async_copy(
                x_hbm.at[pl.ds(base, CH)], bufs[0], sems[0])
            for c in range(NCH):
                if c + 1 < NCH:
                    nb = (c + 1) % 2
                    handles[nb] = pltpu.async_copy(
                        x_hbm.at[pl.ds(base + (c + 1) * CH, CH)],
                        bufs[nb], sems[nb])
                handles[c % 2].wait()
                buf = bufs[c % 2]

                @plsc.parallel_loop(0, CH, L, unroll=8)
                def _body(off):
                    v = buf[pl.ds(off, L)]
                    idx = (v * float(_BINS)).astype(jnp.int32)
                    idx = plsc.bitcast(
                        jnp.minimum(plsc.bitcast(idx, jnp.uint32),
                                    jnp.uint32(_BINS - 1)),
                        jnp.int32)
                    addr = (idx << 4) + lane
                    plsc.addupdate_scatter(hist_v, [addr], ones)

            bin_base = lane << 4
            for j in range(_BINS // L):
                acc = zeros
                for l in range(L):
                    g_idx = bin_base + (j * L * L + l)
                    acc = acc + plsc.load_gather(hist_v, [g_idx])
                row_v[pl.ds(j * L, L)] = acc
            pltpu.sync_copy(row_v, out_hbm.at[b])

    return hist_kernel(xflat)


def _head_body(h_ref, w1, b1, w2, b2, w3, b3, w4, b4, val_ref, w_ref):
    h = h_ref[...]
    mx = jnp.max(h, axis=1, keepdims=True)
    iota = lax.broadcasted_iota(jnp.int32, h.shape, 1)
    amax = jnp.min(jnp.where(h == mx, iota, _BINS), axis=1, keepdims=True)
    cond = jnp.logical_or(iota < amax, h > mx * _HEIGHT_RATE)
    first0 = jnp.min(jnp.where(cond, _BINS, iota), axis=1, keepdims=True)
    val = jnp.where(first0 == _BINS, 0, first0).astype(jnp.float32)
    val_ref[...] = val * (1.0 / _BINS)

    dot = functools.partial(
        jax.lax.dot_general,
        dimension_numbers=(((1,), (0,)), ((), ())),
        preferred_element_type=jnp.float32,
    )
    t = jnp.maximum(dot(h, w1[...]) + b1[...], 0.0)
    t = jnp.maximum(dot(t, w2[...]) + b2[...], 0.0)
    t = jnp.maximum(dot(t, w3[...]) + b3[...], 0.0)
    w_ref[...] = dot(t, w4[...]) + b4[...]


def _head(hist, W1, b1, W2, b2, W3, b3, W4, b4):
    B = hist.shape[0]
    val, w = pl.pallas_call(
        _head_body,
        out_shape=[
            jax.ShapeDtypeStruct((B, 1), jnp.float32),
            jax.ShapeDtypeStruct((B, 1), jnp.float32),
        ],
    )(hist, W1, b1.reshape(1, -1), W2, b2.reshape(1, -1),
      W3, b3.reshape(1, -1), W4, b4.reshape(1, -1))
    return val, w


def _scale_body(x_ref, w_ref, o_ref):
    o_ref[...] = x_ref[...] * w_ref[...]


def _scale(x3, w3d, BR):
    B, R, C = x3.shape
    return pl.pallas_call(
        _scale_body,
        grid=(B, R // BR),
        in_specs=[
            pl.BlockSpec((1, BR, C), lambda b, c: (b, c, 0)),
            pl.BlockSpec((1, 1, 1), lambda b, c: (b, 0, 0)),
        ],
        out_specs=pl.BlockSpec((1, BR, C), lambda b, c: (b, c, 0)),
        out_shape=jax.ShapeDtypeStruct((B, R, C), jnp.float32),
    )(x3, w3d)


def kernel(x, W1, b1, W2, b2, W3, b3, W4, b4):
    B = x.shape[0]
    F = x.size // B
    xflat = x.reshape(-1)

    hist = _sc_histogram(xflat, B, F)
    val, w = _head(hist, W1, b1, W2, b2, W3, b3, W4, b4)

    x3 = x.reshape(B, F // 128, 128)
    out = _scale(x3, w.reshape(B, 1, 1), 2048)
    return out.reshape(x.shape), val.reshape(B)
